# baseline (speedup 1.0000x reference)
import functools
import math

import jax
import jax.numpy as jnp
from jax.experimental import pallas as pl
from jax.experimental.pallas import tpu as pltpu
from jax.experimental.pallas import tpu_sc as plsc

GAIN_MIN = 0.1
GAIN_MAX = 2.0
SR = 16000
LOG_MIN_W = math.log(2.0 * math.pi * 20.0 / SR)
LOG_MAX_W = math.log(math.pi)
LOG_MIN_Q = math.log(0.0707)
LOG_MAX_Q = math.log(2.0)

NSEG = 64
NLANE = 16
CHUNK_L = 128
CHUNK_K = 32


def _sc_segmean(seg, logits_t):
    B, T = seg.shape
    mesh = plsc.VectorSubcoreMesh(core_axis_name="c", subcore_axis_name="s")

    @functools.partial(
        pl.kernel, mesh=mesh,
        compiler_params=pltpu.CompilerParams(needs_layout_passes=False),
        out_type=jax.ShapeDtypeStruct((3, B, T), jnp.float32),
        scratch_types=[
            pltpu.VMEM((T,), jnp.int32),
        ] + [pltpu.VMEM((T,), jnp.float32)] * 3
        + [pltpu.VMEM((T,), jnp.float32)] * 3
        + [pltpu.VMEM((NLANE * NSEG,), jnp.float32)] * 4
        + [pltpu.VMEM((NSEG,), jnp.float32)] * 3,
    )
    def k(seg_hbm, logits_hbm, out_hbm, ids_v,
          lg0, lg1, lg2, out0, out1, out2,
          acc0, acc1, acc2, acc3, mn0, mn1, mn2):
        cid = jax.lax.axis_index("c")
        sid = jax.lax.axis_index("s")
        wid = sid * 2 + cid
        lgs = (lg0, lg1, lg2)
        outs = (out0, out1, out2)
        accs = (acc0, acc1, acc2, acc3)
        mns = (mn0, mn1, mn2)

        @pl.when(wid < B)
        def _():
            b = wid
            pltpu.sync_copy(seg_hbm.at[b], ids_v)
            for ch in range(3):
                pltpu.sync_copy(logits_hbm.at[ch, b], lgs[ch])

            zero16 = jnp.zeros((NLANE,), jnp.float32)
            for ch in range(4):
                for g in range(NLANE * NSEG // NLANE):
                    accs[ch][pl.ds(g * NLANE, NLANE)] = zero16

            lane_iota = jax.lax.iota(jnp.int32, NLANE)
            ones16 = jnp.ones((NLANE,), jnp.float32)

            def scatter_body(i, carry):
                off = i * NLANE
                ids16 = ids_v[pl.ds(off, NLANE)]
                sidx = ids16 * NLANE + lane_iota
                for ch in range(3):
                    x = lgs[ch][pl.ds(off, NLANE)]
                    plsc.addupdate_scatter(accs[ch], [sidx], x)
                plsc.addupdate_scatter(accs[3], [sidx], ones16)
                return carry

            jax.lax.fori_loop(0, T // NLANE, scatter_body, 0)

            for g in range(NSEG // NLANE):
                segv = lane_iota + g * NLANE
                base = segv * NLANE
                cnt = plsc.load_gather(accs[3], [base])
                for lane in range(1, NLANE):
                    cnt = cnt + plsc.load_gather(accs[3], [base + lane])
                cnt = jnp.maximum(cnt, 1.0)
                for ch in range(3):
                    tot = plsc.load_gather(accs[ch], [base])
                    for lane in range(1, NLANE):
                        tot = tot + plsc.load_gather(accs[ch], [base + lane])
                    mns[ch][pl.ds(g * NLANE, NLANE)] = tot / cnt

            def gather_body(i, carry):
                off = i * NLANE
                ids16 = ids_v[pl.ds(off, NLANE)]
                for ch in range(3):
                    m = plsc.load_gather(mns[ch], [ids16])
                    outs[ch][pl.ds(off, NLANE)] = m
                return carry

            jax.lax.fori_loop(0, T // NLANE, gather_body, 0)

            for ch in range(3):
                pltpu.sync_copy(outs[ch], out_hbm.at[ch, b])

    return k(seg, logits_t)


def _tc_kernel(noise_ref, planes_ref, y_ref,
               sf_ref, sa1_ref, sa2_ref, su_ref, sv_ref, sd_ref):
    B, T = noise_ref.shape
    L, K = CHUNK_L, CHUNK_K
    KB = B * K

    gain = GAIN_MIN + (GAIN_MAX - GAIN_MIN) * jax.nn.sigmoid(planes_ref[0])
    w = jnp.exp(LOG_MIN_W + jax.nn.sigmoid(planes_ref[1]) * (LOG_MAX_W - LOG_MIN_W))
    qinv = jnp.exp(-LOG_MIN_Q - jax.nn.sigmoid(planes_ref[2]) * (LOG_MAX_Q - LOG_MIN_Q))
    cosw = jnp.cos(w)
    alpha = jnp.sin(w) * 0.5 * qinv
    inv_a0 = 1.0 / (1.0 + alpha)
    omc = 1.0 - cosw
    b0 = 0.5 * omc * inv_a0
    b1 = omc * inv_a0
    a1c = -2.0 * cosw * inv_a0
    a2c = (1.0 - alpha) * inv_a0

    x = noise_ref[:, :] * gain
    zc = jnp.zeros((B, 1), jnp.float32)
    x1 = jnp.concatenate([zc, x[:, :-1]], axis=1)
    x2 = jnp.concatenate([zc, zc, x[:, :-2]], axis=1)
    fv = b0 * (x + x2) + b1 * x1

    for b in range(B):
        cs = slice(b * K, (b + 1) * K)
        sf_ref[:, cs] = jnp.transpose(fv[b:b + 1, :].reshape(K, L))
        sa1_ref[:, cs] = jnp.transpose(a1c[b:b + 1, :].reshape(K, L))
        sa2_ref[:, cs] = jnp.transpose(a2c[b:b + 1, :].reshape(K, L))

    ones = jnp.ones((1, KB), jnp.float32)
    zeros = jnp.zeros((1, KB), jnp.float32)
    u1, u2, v1, v2, d1, d2 = ones, zeros, zeros, ones, zeros, zeros
    for l in range(L):
        a1 = sa1_ref[l:l + 1, :]
        a2 = sa2_ref[l:l + 1, :]
        fl = sf_ref[l:l + 1, :]
        u = -a1 * u1 - a2 * u2
        v = -a1 * v1 - a2 * v2
        d = fl - a1 * d1 - a2 * d2
        su_ref[l:l + 1, :] = u
        sv_ref[l:l + 1, :] = v
        sd_ref[l:l + 1, :] = d
        u1, u2, v1, v2, d1, d2 = u, u1, v, v1, d, d1

    m00 = su_ref[L - 1:L, :]
    m01 = sv_ref[L - 1:L, :]
    m10 = su_ref[L - 2:L - 1, :]
    m11 = sv_ref[L - 2:L - 1, :]
    q0 = sd_ref[L - 1:L, :]
    q1 = sd_ref[L - 2:L - 1, :]

    kidx = jax.lax.rem(jax.lax.broadcasted_iota(jnp.int32, (1, KB), 1),
                       jnp.int32(K))

    def shift_k(arr, d, fill):
        pad = jnp.full((1, d), fill, jnp.float32)
        rolled = jnp.concatenate([pad, arr[:, :-d]], axis=1)
        return jnp.where(kidx >= d, rolled, fill)

    d = 1
    while d < K:
        s00 = shift_k(m00, d, 1.0)
        s01 = shift_k(m01, d, 0.0)
        s10 = shift_k(m10, d, 0.0)
        s11 = shift_k(m11, d, 1.0)
        t0 = shift_k(q0, d, 0.0)
        t1 = shift_k(q1, d, 0.0)
        n00 = m00 * s00 + m01 * s10
        n01 = m00 * s01 + m01 * s11
        n10 = m10 * s00 + m11 * s10
        n11 = m10 * s01 + m11 * s11
        nq0 = m00 * t0 + m01 * t1 + q0
        nq1 = m10 * t0 + m11 * t1 + q1
        m00, m01, m10, m11, q0, q1 = n00, n01, n10, n11, nq0, nq1
        d *= 2

    y1_all = shift_k(q0, 1, 0.0)
    y2_all = shift_k(q1, 1, 0.0)

    y = su_ref[:, :] * y1_all + sv_ref[:, :] * y2_all + sd_ref[:, :]
    for b in range(B):
        yb = jnp.transpose(y[:, b * K:(b + 1) * K])
        y_ref[b:b + 1, :] = yb.reshape(1, T)


def kernel(noise_bursts, segment_ids, logits):
    B, T = noise_bursts.shape
    seg = segment_ids.astype(jnp.int32)
    logits_t = jnp.transpose(logits, (2, 0, 1))

    planes = _sc_segmean(seg, logits_t)

    return pl.pallas_call(
        _tc_kernel,
        out_shape=jax.ShapeDtypeStruct((B, T), jnp.float32),
        scratch_shapes=[pltpu.VMEM((CHUNK_L, B * CHUNK_K), jnp.float32)] * 6,
    )(noise_bursts, planes)


# speedup vs baseline: 1.1040x; 1.1040x over previous
import functools
import math

import jax
import jax.numpy as jnp
from jax.experimental import pallas as pl
from jax.experimental.pallas import tpu as pltpu
from jax.experimental.pallas import tpu_sc as plsc

GAIN_MIN = 0.1
GAIN_MAX = 2.0
SR = 16000
LOG_MIN_W = math.log(2.0 * math.pi * 20.0 / SR)
LOG_MAX_W = math.log(math.pi)
LOG_MIN_Q = math.log(0.0707)
LOG_MAX_Q = math.log(2.0)

NSEG = 64
NLANE = 16
CHUNK_L = 128
CHUNK_K = 32


def _sc_segmean(seg, logits_t):
    B, T = seg.shape
    mesh = plsc.VectorSubcoreMesh(core_axis_name="c", subcore_axis_name="s")

    H = T // 2

    @functools.partial(
        pl.kernel, mesh=mesh,
        compiler_params=pltpu.CompilerParams(needs_layout_passes=False),
        out_type=jax.ShapeDtypeStruct((3, B, T), jnp.float32),
        scratch_types=[
            pltpu.VMEM((H,), jnp.int32),
        ] + [pltpu.VMEM((H,), jnp.float32)] * 3
        + [pltpu.VMEM((H,), jnp.float32)] * 3
        + [pltpu.VMEM((NLANE * NSEG,), jnp.float32)] * 4
        + [pltpu.VMEM((4, NSEG), jnp.float32)] * 2
        + [pltpu.VMEM((NSEG,), jnp.float32)] * 3
        + [pltpu.VMEM_SHARED((NLANE, 4, NSEG), jnp.float32)],
    )
    def k(seg_hbm, logits_hbm, out_hbm, ids_v,
          lg0, lg1, lg2, out0, out1, out2,
          acc0, acc1, acc2, acc3, part_v, full_v, mn0, mn1, mn2, shared):
        cid = jax.lax.axis_index("c")
        sid = jax.lax.axis_index("s")
        slot = sid // 2
        half = sid % 2
        b = cid * (B // 2) + slot
        t0 = half * H

        pltpu.sync_copy(seg_hbm.at[b, pl.ds(t0, H)], ids_v)
        lgs = (lg0, lg1, lg2)
        outs = (out0, out1, out2)
        accs = (acc0, acc1, acc2, acc3)
        mns = (mn0, mn1, mn2)
        for ch in range(3):
            pltpu.sync_copy(logits_hbm.at[ch, b, pl.ds(t0, H)], lgs[ch])

        zero16 = jnp.zeros((NLANE,), jnp.float32)
        for ch in range(4):
            for g in range(NSEG):
                accs[ch][pl.ds(g * NLANE, NLANE)] = zero16

        lane_iota = jax.lax.iota(jnp.int32, NLANE)
        ones16 = jnp.ones((NLANE,), jnp.float32)

        def scatter_body(i, carry):
            off = i * NLANE
            ids16 = ids_v[pl.ds(off, NLANE)]
            sidx = ids16 * NLANE + lane_iota
            for ch in range(3):
                x = lgs[ch][pl.ds(off, NLANE)]
                plsc.addupdate_scatter(accs[ch], [sidx], x)
            plsc.addupdate_scatter(accs[3], [sidx], ones16)
            return carry

        jax.lax.fori_loop(0, H // NLANE, scatter_body, 0)

        for g in range(NSEG // NLANE):
            base = (lane_iota + g * NLANE) * NLANE
            for ch in range(4):
                tot = plsc.load_gather(accs[ch], [base])
                for lane in range(1, NLANE):
                    tot = tot + plsc.load_gather(accs[ch], [base + lane])
                part_v[ch, pl.ds(g * NLANE, NLANE)] = tot

        pltpu.sync_copy(part_v, shared.at[sid])
        plsc.subcore_barrier()
        pltpu.sync_copy(shared.at[sid + 1 - 2 * half], full_v)

        for g in range(NSEG // NLANE):
            sl = pl.ds(g * NLANE, NLANE)
            cnt = jnp.maximum(full_v[3, sl] + part_v[3, sl], 1.0)
            for ch in range(3):
                mns[ch][sl] = (full_v[ch, sl] + part_v[ch, sl]) / cnt

        def gather_body(i, carry):
            off = i * NLANE
            ids16 = ids_v[pl.ds(off, NLANE)]
            for ch in range(3):
                m = plsc.load_gather(mns[ch], [ids16])
                outs[ch][pl.ds(off, NLANE)] = m
            return carry

        jax.lax.fori_loop(0, H // NLANE, gather_body, 0)

        for ch in range(3):
            pltpu.sync_copy(outs[ch], out_hbm.at[ch, b, pl.ds(t0, H)])

    return k(seg, logits_t)


def _tc_kernel(noise_ref, planes_ref, y_ref,
               sf_ref, sa1_ref, sa2_ref, su_ref, sv_ref, sd_ref):
    B, T = noise_ref.shape
    L, K = CHUNK_L, CHUNK_K
    KB = B * K

    gain = GAIN_MIN + (GAIN_MAX - GAIN_MIN) * jax.nn.sigmoid(planes_ref[0])
    w = jnp.exp(LOG_MIN_W + jax.nn.sigmoid(planes_ref[1]) * (LOG_MAX_W - LOG_MIN_W))
    qinv = jnp.exp(-LOG_MIN_Q - jax.nn.sigmoid(planes_ref[2]) * (LOG_MAX_Q - LOG_MIN_Q))
    cosw = jnp.cos(w)
    alpha = jnp.sin(w) * 0.5 * qinv
    inv_a0 = 1.0 / (1.0 + alpha)
    omc = 1.0 - cosw
    b0 = 0.5 * omc * inv_a0
    b1 = omc * inv_a0
    a1c = -2.0 * cosw * inv_a0
    a2c = (1.0 - alpha) * inv_a0

    x = noise_ref[:, :] * gain
    zc = jnp.zeros((B, 1), jnp.float32)
    x1 = jnp.concatenate([zc, x[:, :-1]], axis=1)
    x2 = jnp.concatenate([zc, zc, x[:, :-2]], axis=1)
    fv = b0 * (x + x2) + b1 * x1

    for b in range(B):
        cs = slice(b * K, (b + 1) * K)
        sf_ref[:, cs] = jnp.transpose(fv[b:b + 1, :].reshape(K, L))
        sa1_ref[:, cs] = jnp.transpose(a1c[b:b + 1, :].reshape(K, L))
        sa2_ref[:, cs] = jnp.transpose(a2c[b:b + 1, :].reshape(K, L))

    ones = jnp.ones((1, KB), jnp.float32)
    zeros = jnp.zeros((1, KB), jnp.float32)
    u1, u2, v1, v2, d1, d2 = ones, zeros, zeros, ones, zeros, zeros
    for l in range(L):
        a1 = sa1_ref[l:l + 1, :]
        a2 = sa2_ref[l:l + 1, :]
        fl = sf_ref[l:l + 1, :]
        u = -a1 * u1 - a2 * u2
        v = -a1 * v1 - a2 * v2
        d = fl - a1 * d1 - a2 * d2
        su_ref[l:l + 1, :] = u
        sv_ref[l:l + 1, :] = v
        sd_ref[l:l + 1, :] = d
        u1, u2, v1, v2, d1, d2 = u, u1, v, v1, d, d1

    m00 = su_ref[L - 1:L, :]
    m01 = sv_ref[L - 1:L, :]
    m10 = su_ref[L - 2:L - 1, :]
    m11 = sv_ref[L - 2:L - 1, :]
    q0 = sd_ref[L - 1:L, :]
    q1 = sd_ref[L - 2:L - 1, :]

    kidx = jax.lax.rem(jax.lax.broadcasted_iota(jnp.int32, (1, KB), 1),
                       jnp.int32(K))

    def shift_k(arr, d, fill):
        pad = jnp.full((1, d), fill, jnp.float32)
        rolled = jnp.concatenate([pad, arr[:, :-d]], axis=1)
        return jnp.where(kidx >= d, rolled, fill)

    d = 1
    while d < K:
        s00 = shift_k(m00, d, 1.0)
        s01 = shift_k(m01, d, 0.0)
        s10 = shift_k(m10, d, 0.0)
        s11 = shift_k(m11, d, 1.0)
        t0 = shift_k(q0, d, 0.0)
        t1 = shift_k(q1, d, 0.0)
        n00 = m00 * s00 + m01 * s10
        n01 = m00 * s01 + m01 * s11
        n10 = m10 * s00 + m11 * s10
        n11 = m10 * s01 + m11 * s11
        nq0 = m00 * t0 + m01 * t1 + q0
        nq1 = m10 * t0 + m11 * t1 + q1
        m00, m01, m10, m11, q0, q1 = n00, n01, n10, n11, nq0, nq1
        d *= 2

    y1_all = shift_k(q0, 1, 0.0)
    y2_all = shift_k(q1, 1, 0.0)

    y = su_ref[:, :] * y1_all + sv_ref[:, :] * y2_all + sd_ref[:, :]
    for b in range(B):
        yb = jnp.transpose(y[:, b * K:(b + 1) * K])
        y_ref[b:b + 1, :] = yb.reshape(1, T)


def kernel(noise_bursts, segment_ids, logits):
    B, T = noise_bursts.shape
    seg = segment_ids.astype(jnp.int32)
    logits_t = jnp.transpose(logits, (2, 0, 1))

    planes = _sc_segmean(seg, logits_t)

    return pl.pallas_call(
        _tc_kernel,
        out_shape=jax.ShapeDtypeStruct((B, T), jnp.float32),
        scratch_shapes=[pltpu.VMEM((CHUNK_L, B * CHUNK_K), jnp.float32)] * 6,
    )(noise_bursts, planes)


# speedup vs baseline: 1.1091x; 1.0047x over previous
import functools
import math

import jax
import jax.numpy as jnp
from jax.experimental import pallas as pl
from jax.experimental.pallas import tpu as pltpu
from jax.experimental.pallas import tpu_sc as plsc

GAIN_MIN = 0.1
GAIN_MAX = 2.0
SR = 16000
LOG_MIN_W = math.log(2.0 * math.pi * 20.0 / SR)
LOG_MAX_W = math.log(math.pi)
LOG_MIN_Q = math.log(0.0707)
LOG_MAX_Q = math.log(2.0)

NSEG = 64
NLANE = 16
CHUNK_L = 128
CHUNK_K = 32


def _sc_segmean(seg, logits_t):
    B, T = seg.shape
    mesh = plsc.VectorSubcoreMesh(core_axis_name="c", subcore_axis_name="s")

    @functools.partial(
        pl.kernel, mesh=mesh,
        compiler_params=pltpu.CompilerParams(needs_layout_passes=False),
        out_type=jax.ShapeDtypeStruct((3, B, T), jnp.float32),
        scratch_types=[
            pltpu.VMEM((T,), jnp.int32),
        ] + [pltpu.VMEM((T,), jnp.float32)] * 2
        + [pltpu.VMEM((T,), jnp.float32)] * 2
        + [pltpu.VMEM((NLANE * NSEG,), jnp.float32)] * 2
        + [pltpu.VMEM((2, NSEG), jnp.float32)] * 2
        + [pltpu.VMEM((NSEG,), jnp.float32)] * 2
        + [pltpu.VMEM_SHARED((NLANE, 2, NSEG), jnp.float32)],
    )
    def k(seg_hbm, logits_hbm, out_hbm, ids_v,
          lga, lgb, outa, outb, acca, accb, part_v, full_v, mna, mnb, shared):
        cid = jax.lax.axis_index("c")
        sid = jax.lax.axis_index("s")
        slot = sid // 2
        role = sid % 2
        b = cid * (B // 2) + slot

        pltpu.sync_copy(seg_hbm.at[b], ids_v)
        lane_iota = jax.lax.iota(jnp.int32, NLANE)
        ones16 = jnp.ones((NLANE,), jnp.float32)
        zero16 = jnp.zeros((NLANE,), jnp.float32)

        @pl.when(role == 0)
        def _():
            pltpu.sync_copy(logits_hbm.at[0, b], lga)
            pltpu.sync_copy(logits_hbm.at[1, b], lgb)

        @pl.when(role == 1)
        def _():
            pltpu.sync_copy(logits_hbm.at[2, b], lga)

        for g in range(NSEG):
            acca[pl.ds(g * NLANE, NLANE)] = zero16
            accb[pl.ds(g * NLANE, NLANE)] = zero16

        @pl.when(role == 0)
        def _():
            def scatter_a(i, carry):
                off = i * NLANE
                ids16 = ids_v[pl.ds(off, NLANE)]
                sidx = ids16 * NLANE + lane_iota
                plsc.addupdate_scatter(acca, [sidx], lga[pl.ds(off, NLANE)])
                plsc.addupdate_scatter(accb, [sidx], lgb[pl.ds(off, NLANE)])
                return carry
            jax.lax.fori_loop(0, T // NLANE, scatter_a, 0)

        @pl.when(role == 1)
        def _():
            def scatter_b(i, carry):
                off = i * NLANE
                ids16 = ids_v[pl.ds(off, NLANE)]
                sidx = ids16 * NLANE + lane_iota
                plsc.addupdate_scatter(acca, [sidx], lga[pl.ds(off, NLANE)])
                plsc.addupdate_scatter(accb, [sidx], ones16)
                return carry
            jax.lax.fori_loop(0, T // NLANE, scatter_b, 0)

        for g in range(NSEG // NLANE):
            base = (lane_iota + g * NLANE) * NLANE
            for ci, acc in enumerate((acca, accb)):
                tot = plsc.load_gather(acc, [base])
                for lane in range(1, NLANE):
                    tot = tot + plsc.load_gather(acc, [base + lane])
                part_v[ci, pl.ds(g * NLANE, NLANE)] = tot

        @pl.when(role == 1)
        def _():
            pltpu.sync_copy(part_v, shared.at[sid])
        plsc.subcore_barrier()

        @pl.when(role == 0)
        def _():
            pltpu.sync_copy(shared.at[sid + 1], full_v)
            for g in range(NSEG // NLANE):
                sl = pl.ds(g * NLANE, NLANE)
                cnt = jnp.maximum(full_v[1, sl], 1.0)
                mna[sl] = part_v[0, sl] / cnt
                mnb[sl] = part_v[1, sl] / cnt

            def gather_a(i, carry):
                off = i * NLANE
                ids16 = ids_v[pl.ds(off, NLANE)]
                outa[pl.ds(off, NLANE)] = plsc.load_gather(mna, [ids16])
                outb[pl.ds(off, NLANE)] = plsc.load_gather(mnb, [ids16])
                return carry
            jax.lax.fori_loop(0, T // NLANE, gather_a, 0)
            pltpu.sync_copy(outa, out_hbm.at[0, b])
            pltpu.sync_copy(outb, out_hbm.at[1, b])

        @pl.when(role == 1)
        def _():
            for g in range(NSEG // NLANE):
                sl = pl.ds(g * NLANE, NLANE)
                cnt = jnp.maximum(part_v[1, sl], 1.0)
                mna[sl] = part_v[0, sl] / cnt

            def gather_b(i, carry):
                off = i * NLANE
                ids16 = ids_v[pl.ds(off, NLANE)]
                outa[pl.ds(off, NLANE)] = plsc.load_gather(mna, [ids16])
                return carry
            jax.lax.fori_loop(0, T // NLANE, gather_b, 0)
            pltpu.sync_copy(outa, out_hbm.at[2, b])

    return k(seg, logits_t)


def _tc_kernel(noise_ref, planes_ref, y_ref,
               sf_ref, sa1_ref, sa2_ref, su_ref, sv_ref, sd_ref):
    B, T = noise_ref.shape
    L, K = CHUNK_L, CHUNK_K
    KB = B * K

    gain = GAIN_MIN + (GAIN_MAX - GAIN_MIN) * jax.nn.sigmoid(planes_ref[0])
    w = jnp.exp(LOG_MIN_W + jax.nn.sigmoid(planes_ref[1]) * (LOG_MAX_W - LOG_MIN_W))
    qinv = jnp.exp(-LOG_MIN_Q - jax.nn.sigmoid(planes_ref[2]) * (LOG_MAX_Q - LOG_MIN_Q))
    cosw = jnp.cos(w)
    alpha = jnp.sin(w) * 0.5 * qinv
    inv_a0 = 1.0 / (1.0 + alpha)
    omc = 1.0 - cosw
    b0 = 0.5 * omc * inv_a0
    b1 = omc * inv_a0
    a1c = -2.0 * cosw * inv_a0
    a2c = (1.0 - alpha) * inv_a0

    x = noise_ref[:, :] * gain
    zc = jnp.zeros((B, 1), jnp.float32)
    x1 = jnp.concatenate([zc, x[:, :-1]], axis=1)
    x2 = jnp.concatenate([zc, zc, x[:, :-2]], axis=1)
    fv = b0 * (x + x2) + b1 * x1

    for b in range(B):
        cs = slice(b * K, (b + 1) * K)
        sf_ref[:, cs] = jnp.transpose(fv[b:b + 1, :].reshape(K, L))
        sa1_ref[:, cs] = jnp.transpose(a1c[b:b + 1, :].reshape(K, L))
        sa2_ref[:, cs] = jnp.transpose(a2c[b:b + 1, :].reshape(K, L))

    ones = jnp.ones((1, KB), jnp.float32)
    zeros = jnp.zeros((1, KB), jnp.float32)
    u1, u2, v1, v2, d1, d2 = ones, zeros, zeros, ones, zeros, zeros
    for l in range(L):
        a1 = sa1_ref[l:l + 1, :]
        a2 = sa2_ref[l:l + 1, :]
        fl = sf_ref[l:l + 1, :]
        u = -a1 * u1 - a2 * u2
        v = -a1 * v1 - a2 * v2
        d = fl - a1 * d1 - a2 * d2
        su_ref[l:l + 1, :] = u
        sv_ref[l:l + 1, :] = v
        sd_ref[l:l + 1, :] = d
        u1, u2, v1, v2, d1, d2 = u, u1, v, v1, d, d1

    m00 = su_ref[L - 1:L, :]
    m01 = sv_ref[L - 1:L, :]
    m10 = su_ref[L - 2:L - 1, :]
    m11 = sv_ref[L - 2:L - 1, :]
    q0 = sd_ref[L - 1:L, :]
    q1 = sd_ref[L - 2:L - 1, :]

    kidx = jax.lax.rem(jax.lax.broadcasted_iota(jnp.int32, (1, KB), 1),
                       jnp.int32(K))

    def shift_k(arr, d, fill):
        pad = jnp.full((1, d), fill, jnp.float32)
        rolled = jnp.concatenate([pad, arr[:, :-d]], axis=1)
        return jnp.where(kidx >= d, rolled, fill)

    d = 1
    while d < K:
        s00 = shift_k(m00, d, 1.0)
        s01 = shift_k(m01, d, 0.0)
        s10 = shift_k(m10, d, 0.0)
        s11 = shift_k(m11, d, 1.0)
        t0 = shift_k(q0, d, 0.0)
        t1 = shift_k(q1, d, 0.0)
        n00 = m00 * s00 + m01 * s10
        n01 = m00 * s01 + m01 * s11
        n10 = m10 * s00 + m11 * s10
        n11 = m10 * s01 + m11 * s11
        nq0 = m00 * t0 + m01 * t1 + q0
        nq1 = m10 * t0 + m11 * t1 + q1
        m00, m01, m10, m11, q0, q1 = n00, n01, n10, n11, nq0, nq1
        d *= 2

    y1_all = shift_k(q0, 1, 0.0)
    y2_all = shift_k(q1, 1, 0.0)

    y = su_ref[:, :] * y1_all + sv_ref[:, :] * y2_all + sd_ref[:, :]
    for b in range(B):
        yb = jnp.transpose(y[:, b * K:(b + 1) * K])
        y_ref[b:b + 1, :] = yb.reshape(1, T)


def kernel(noise_bursts, segment_ids, logits):
    B, T = noise_bursts.shape
    seg = segment_ids.astype(jnp.int32)
    logits_t = jnp.transpose(logits, (2, 0, 1))

    planes = _sc_segmean(seg, logits_t)

    return pl.pallas_call(
        _tc_kernel,
        out_shape=jax.ShapeDtypeStruct((B, T), jnp.float32),
        scratch_shapes=[pltpu.VMEM((CHUNK_L, B * CHUNK_K), jnp.float32)] * 6,
    )(noise_bursts, planes)
